# Initial kernel scaffold; baseline (speedup 1.0000x reference)
import jax
import jax.numpy as jnp
from jax.experimental import pallas as pl


def kernel(x, edge_index, w, W_l, b_l, W_r, b_r):
    raise NotImplementedError("write your pallas kernel here")


# speedup vs baseline: 3.1520x; 3.1520x over previous
import functools

import jax
import jax.numpy as jnp
from jax import lax
from jax.experimental import pallas as pl
from jax.experimental.pallas import tpu as pltpu
from jax.experimental.pallas import tpu_sc as plsc

N = 10000
NPAD = 10240
D = 128
E = 320000
EPAD = 323584
PERW = EPAD // 32
K = 128
NCHUNK = PERW // K
ROWS_PER_SUB = NPAD // 16


def _zn_body(x_ref, wt_ref, b_ref, o_ref):
    h = jnp.dot(x_ref[...], wt_ref[...], preferred_element_type=jnp.float32)
    o_ref[...] = jnp.maximum(h + b_ref[...], 0.0)


def _zn_call(xp, WlT, b_l):
    return pl.pallas_call(
        _zn_body,
        grid=(NPAD // 1024,),
        in_specs=[
            pl.BlockSpec((1024, D), lambda i: (i, 0)),
            pl.BlockSpec((D, D), lambda i: (0, 0)),
            pl.BlockSpec((1, D), lambda i: (0, 0)),
        ],
        out_specs=pl.BlockSpec((1024, D), lambda i: (i, 0)),
        out_shape=jax.ShapeDtypeStruct((NPAD, D), jnp.float32),
    )(xp, WlT, b_l)


_mesh = plsc.VectorSubcoreMesh(core_axis_name="c", subcore_axis_name="s")


@functools.partial(
    pl.kernel,
    out_type=(
        jax.ShapeDtypeStruct((2, NPAD, D), jnp.float32),
        jax.ShapeDtypeStruct((2, NPAD), jnp.float32),
    ),
    mesh=_mesh,
    scratch_types=[
        pltpu.VMEM((K,), jnp.int32),
        pltpu.VMEM((K,), jnp.int32),
        pltpu.VMEM((K,), jnp.float32),
        pltpu.VMEM((K, 16), jnp.float32),
        pltpu.VMEM((K, D), jnp.float32),
        pltpu.VMEM((ROWS_PER_SUB,), jnp.float32),
        pltpu.VMEM_SHARED((NPAD, D), jnp.float32),
        pltpu.VMEM_SHARED((NPAD,), jnp.float32),
        pltpu.SemaphoreType.DMA,
    ],
)
def _sc_agg(zn_hbm, src_hbm, dst_hbm, w_hbm, w16_hbm, p_hbm, pw_hbm,
            isrc, idst, wbuf, wbuf16, rows, wz, acc, accw, sem):
    cid = lax.axis_index("c")
    sid = lax.axis_index("s")
    wid = cid * 16 + sid
    zv = jnp.zeros((16,), jnp.float32)

    def _zrow(k, carry):
        for j in range(8):
            rows[k, pl.ds(j * 16, 16)] = zv
        return carry

    lax.fori_loop(0, K, _zrow, 0)

    def _zwz(k, carry):
        wz[pl.ds(k * 16, 16)] = zv
        return carry

    lax.fori_loop(0, ROWS_PER_SUB // 16, _zwz, 0)

    row0 = sid * ROWS_PER_SUB
    for t in range(ROWS_PER_SUB // K):
        pltpu.sync_copy(rows, acc.at[pl.ds(row0 + t * K, K)])
    pltpu.sync_copy(wz, accw.at[pl.ds(row0, ROWS_PER_SUB)])
    plsc.subcore_barrier()

    base = wid * PERW

    def _chunk(c, carry):
        off = base + c * K
        pltpu.sync_copy(src_hbm.at[pl.ds(off, K)], isrc)
        pltpu.sync_copy(dst_hbm.at[pl.ds(off, K)], idst)
        pltpu.sync_copy(w_hbm.at[pl.ds(off, K)], wbuf)
        pltpu.sync_copy(w16_hbm.at[pl.ds(off, K)], wbuf16)
        pltpu.async_copy(zn_hbm.at[isrc], rows, sem).wait()

        def _scale(k, inner):
            wv = wbuf16[k]
            for j in range(8):
                sl = pl.ds(j * 16, 16)
                rows[k, sl] = rows[k, sl] * wv
            return inner

        lax.fori_loop(0, K, _scale, 0)
        pltpu.sync_copy(rows, acc.at[idst], add=True)
        pltpu.sync_copy(wbuf, accw.at[idst], add=True)
        return carry

    lax.fori_loop(0, NCHUNK, _chunk, 0)
    plsc.subcore_barrier()

    pltpu.sync_copy(acc.at[pl.ds(row0, ROWS_PER_SUB)],
                    p_hbm.at[cid, pl.ds(row0, ROWS_PER_SUB)])
    pltpu.sync_copy(accw.at[pl.ds(row0, ROWS_PER_SUB)],
                    pw_hbm.at[cid, pl.ds(row0, ROWS_PER_SUB)])


def _out_body(x_ref, p_ref, pw_ref, wt_ref, b_ref, o_ref):
    aggw = pw_ref[0] + pw_ref[1] + 1.0
    agg = (p_ref[0] + p_ref[1]) / aggw
    h = jnp.dot(x_ref[...], wt_ref[0:D, :], preferred_element_type=jnp.float32)
    h = h + jnp.dot(agg, wt_ref[D:2 * D, :], preferred_element_type=jnp.float32)
    h = jnp.maximum(h + b_ref[...], 0.0)
    nrm = jnp.sqrt(jnp.sum(h * h, axis=1, keepdims=True))
    o_ref[...] = h / jnp.maximum(nrm, 1e-12)


def _out_call(xp, P, Pw_col, WrT, b_r):
    return pl.pallas_call(
        _out_body,
        grid=(NPAD // 1024,),
        in_specs=[
            pl.BlockSpec((1024, D), lambda i: (i, 0)),
            pl.BlockSpec((2, 1024, D), lambda i: (0, i, 0)),
            pl.BlockSpec((2, 1024, 1), lambda i: (0, i, 0)),
            pl.BlockSpec((2 * D, D), lambda i: (0, 0)),
            pl.BlockSpec((1, D), lambda i: (0, 0)),
        ],
        out_specs=pl.BlockSpec((1024, D), lambda i: (i, 0)),
        out_shape=jax.ShapeDtypeStruct((NPAD, D), jnp.float32),
    )(xp, P, Pw_col, WrT, b_r)


def kernel(x, edge_index, w, W_l, b_l, W_r, b_r):
    xp = jnp.pad(x, ((0, NPAD - N), (0, 0)))
    src = jnp.pad(edge_index[0, :], (0, EPAD - E))
    dst = jnp.pad(edge_index[1, :], (0, EPAD - E), constant_values=NPAD - 1)
    wp = jnp.pad(w, (0, EPAD - E))
    w16 = jnp.broadcast_to(wp[:, None], (EPAD, 16))

    zn = _zn_call(xp, W_l.T, b_l.reshape(1, D))
    P, Pw = _sc_agg(zn, src, dst, wp, w16)
    out = _out_call(xp, P, Pw.reshape(2, NPAD, 1), W_r.T, b_r.reshape(1, D))
    return out[:N]
